# baseline (speedup 1.0000x reference)
import functools

import jax
import jax.numpy as jnp
from jax import lax
from jax.experimental import pallas as pl
from jax.experimental.pallas import tpu as pltpu
from jax.experimental.pallas import tpu_sc as plsc

D = 256
K = 8192
BN = 2048
SB = 256
COMMITMENT_COST_ = 0.25

SC_CORES = 2
SC_SUBCORES = 16
SC_WORKERS = SC_CORES * SC_SUBCORES
GATHER_CHUNK = 128


def _dist_argmin_body(x_ref, cb_ref, esq_ref, iota_ref, idx_ref, loss_ref,
                      acc_ref):
    i = pl.program_id(0)
    nsteps = pl.num_programs(0)

    cb = cb_ref[...]
    esq = esq_ref[...]
    iota = iota_ref[...]

    @pl.when(i == 0)
    def _():
        acc_ref[...] = jnp.zeros((1, 1), jnp.float32)

    for s in range(BN // SB):
        ss = slice(s * SB, (s + 1) * SB)
        x = x_ref[ss, :]
        z_sq = jnp.sum(x * x, axis=1, keepdims=True)
        dots2 = lax.dot_general(
            x * (-2.0), cb, (((1,), (1,)), ((), ())),
            preferred_element_type=jnp.float32)
        dists = dots2 + z_sq + esq
        m = jnp.min(dists, axis=1, keepdims=True)
        idxf = jnp.min(jnp.where(dists == m, iota, jnp.float32(K)),
                       axis=1, keepdims=True)
        idx_ref[ss, :] = idxf.astype(jnp.int32)
        acc_ref[...] += jnp.sum(m).reshape(1, 1)

    @pl.when(i == nsteps - 1)
    def _():
        e_loss = acc_ref[...] / jnp.float32(nsteps * BN * D)
        loss_ref[...] = e_loss + COMMITMENT_COST_ * e_loss


def _tc_dist_argmin(flat, codebook, e_sq, iota_row):
    n = flat.shape[0]
    grid = n // BN
    return pl.pallas_call(
        _dist_argmin_body,
        grid=(grid,),
        in_specs=[
            pl.BlockSpec((BN, D), lambda i: (i, 0)),
            pl.BlockSpec((K, D), lambda i: (0, 0)),
            pl.BlockSpec((1, K), lambda i: (0, 0)),
            pl.BlockSpec((1, K), lambda i: (0, 0)),
        ],
        out_specs=[
            pl.BlockSpec((BN, 1), lambda i: (i, 0)),
            pl.BlockSpec((1, 1), lambda i: (0, 0)),
        ],
        out_shape=[
            jax.ShapeDtypeStruct((n, 1), jnp.int32),
            jax.ShapeDtypeStruct((1, 1), jnp.float32),
        ],
        scratch_shapes=[pltpu.VMEM((1, 1), jnp.float32)],
    )(flat, codebook, e_sq, iota_row)


def _sc_gather(codebook, indices):
    n = indices.shape[0]
    b_per_w = n // SC_WORKERS
    mesh = plsc.VectorSubcoreMesh(core_axis_name="c", subcore_axis_name="s")

    @functools.partial(
        pl.kernel,
        mesh=mesh,
        out_type=jax.ShapeDtypeStruct((n, D), jnp.float32),
        scratch_types=[
            pltpu.VMEM((GATHER_CHUNK,), jnp.int32),
            pltpu.VMEM((GATHER_CHUNK,), jnp.int32),
            pltpu.VMEM((GATHER_CHUNK, D), jnp.float32),
            pltpu.VMEM((GATHER_CHUNK, D), jnp.float32),
            pltpu.SemaphoreType.DMA,
            pltpu.SemaphoreType.DMA,
        ],
    )
    def k(cb_hbm, idx_hbm, out_hbm, idx_v0, idx_v1, rows_v0, rows_v1,
          sem0, sem1):
        wid = lax.axis_index("s") * SC_CORES + lax.axis_index("c")
        base = wid * b_per_w

        @pl.loop(0, b_per_w, step=2 * GATHER_CHUNK)
        def _(off):
            o0 = base + off
            o1 = o0 + GATHER_CHUNK
            pltpu.sync_copy(idx_hbm.at[pl.ds(o0, GATHER_CHUNK)], idx_v0)
            g0 = pltpu.async_copy(cb_hbm.at[idx_v0], rows_v0, sem0)
            pltpu.sync_copy(idx_hbm.at[pl.ds(o1, GATHER_CHUNK)], idx_v1)
            g1 = pltpu.async_copy(cb_hbm.at[idx_v1], rows_v1, sem1)
            g0.wait()
            pltpu.sync_copy(rows_v0, out_hbm.at[pl.ds(o0, GATHER_CHUNK)])
            g1.wait()
            pltpu.sync_copy(rows_v1, out_hbm.at[pl.ds(o1, GATHER_CHUNK)])

    return k(codebook, indices)


def kernel(weights, codebook):
    flat = weights.reshape(-1, D)
    e_sq = jnp.sum(codebook * codebook, axis=1)[None, :]
    iota_row = jnp.arange(K, dtype=jnp.float32)[None, :]
    idx2d, loss = _tc_dist_argmin(flat, codebook, e_sq, iota_row)
    indices = idx2d.reshape(-1)
    quantized = _sc_gather(codebook, indices)
    return quantized.reshape(weights.shape), loss[0, 0]


# speedup vs baseline: 1.0027x; 1.0027x over previous
import functools

import jax
import jax.numpy as jnp
from jax import lax
from jax.experimental import pallas as pl
from jax.experimental.pallas import tpu as pltpu
from jax.experimental.pallas import tpu_sc as plsc

D = 256
K = 8192
BN = 2048
SB = 512
COMMITMENT_COST_ = 0.25

SC_CORES = 2
SC_SUBCORES = 16
SC_WORKERS = SC_CORES * SC_SUBCORES
GATHER_CHUNK = 128


def _dist_argmin_body(x_ref, cb_ref, esq_ref, iota_ref, idx_ref, loss_ref,
                      acc_ref):
    i = pl.program_id(0)
    nsteps = pl.num_programs(0)

    cb = cb_ref[...]
    esq = esq_ref[...]
    iota = iota_ref[...]

    @pl.when(i == 0)
    def _():
        acc_ref[...] = jnp.zeros((1, 1), jnp.float32)

    for s in range(BN // SB):
        ss = slice(s * SB, (s + 1) * SB)
        x = x_ref[ss, :]
        z_sq = jnp.sum(x * x, axis=1, keepdims=True)
        dots2 = lax.dot_general(
            x * (-2.0), cb, (((1,), (1,)), ((), ())),
            preferred_element_type=jnp.float32)
        dists = dots2 + z_sq + esq
        m = jnp.min(dists, axis=1, keepdims=True)
        idxf = jnp.min(jnp.where(dists == m, iota, jnp.float32(K)),
                       axis=1, keepdims=True)
        idx_ref[ss, :] = idxf.astype(jnp.int32)
        acc_ref[...] += jnp.sum(m).reshape(1, 1)

    @pl.when(i == nsteps - 1)
    def _():
        e_loss = acc_ref[...] / jnp.float32(nsteps * BN * D)
        loss_ref[...] = e_loss + COMMITMENT_COST_ * e_loss


def _tc_dist_argmin(flat, codebook, e_sq, iota_row):
    n = flat.shape[0]
    grid = n // BN
    return pl.pallas_call(
        _dist_argmin_body,
        grid=(grid,),
        in_specs=[
            pl.BlockSpec((BN, D), lambda i: (i, 0)),
            pl.BlockSpec((K, D), lambda i: (0, 0)),
            pl.BlockSpec((1, K), lambda i: (0, 0)),
            pl.BlockSpec((1, K), lambda i: (0, 0)),
        ],
        out_specs=[
            pl.BlockSpec((BN, 1), lambda i: (i, 0)),
            pl.BlockSpec((1, 1), lambda i: (0, 0)),
        ],
        out_shape=[
            jax.ShapeDtypeStruct((n, 1), jnp.int32),
            jax.ShapeDtypeStruct((1, 1), jnp.float32),
        ],
        scratch_shapes=[pltpu.VMEM((1, 1), jnp.float32)],
    )(flat, codebook, e_sq, iota_row)


def _sc_gather(codebook, indices):
    n = indices.shape[0]
    b_per_w = n // SC_WORKERS
    mesh = plsc.VectorSubcoreMesh(core_axis_name="c", subcore_axis_name="s")

    nbuf = 4
    ch = GATHER_CHUNK // 2

    @functools.partial(
        pl.kernel,
        mesh=mesh,
        out_type=jax.ShapeDtypeStruct((n, D), jnp.float32),
        scratch_types=(
            [pltpu.VMEM((ch,), jnp.int32)] * nbuf
            + [pltpu.VMEM((ch, D), jnp.float32)] * nbuf
            + [pltpu.SemaphoreType.DMA] * nbuf
        ),
    )
    def k(cb_hbm, idx_hbm, out_hbm, *scratch):
        idx_vs = scratch[:nbuf]
        rows_vs = scratch[nbuf:2 * nbuf]
        sems = scratch[2 * nbuf:]
        wid = lax.axis_index("s") * SC_CORES + lax.axis_index("c")
        base = wid * b_per_w

        @pl.loop(0, b_per_w, step=nbuf * ch)
        def _(off):
            for b in range(nbuf):
                o = base + off + b * ch
                pltpu.sync_copy(idx_hbm.at[pl.ds(o, ch)], idx_vs[b])
                pltpu.async_copy(cb_hbm.at[idx_vs[b]], rows_vs[b], sems[b])
            for b in range(nbuf):
                o = base + off + b * ch
                pltpu.make_async_copy(cb_hbm.at[idx_vs[b]], rows_vs[b],
                                      sems[b]).wait()
                pltpu.sync_copy(rows_vs[b], out_hbm.at[pl.ds(o, ch)])

    return k(codebook, indices)


def kernel(weights, codebook):
    flat = weights.reshape(-1, D)
    e_sq = jnp.sum(codebook * codebook, axis=1)[None, :]
    iota_row = jnp.arange(K, dtype=jnp.float32)[None, :]
    idx2d, loss = _tc_dist_argmin(flat, codebook, e_sq, iota_row)
    indices = idx2d.reshape(-1)
    quantized = _sc_gather(codebook, indices)
    return quantized.reshape(weights.shape), loss[0, 0]
